# baseline (speedup 1.0000x reference)
import functools

import jax
import jax.numpy as jnp
from jax import lax
from jax.experimental import pallas as pl
from jax.experimental.pallas import tpu as pltpu
from jax.experimental.pallas import tpu_sc as plsc

B, S, D = 2, 2048, 1024
F = 2048
E = 8
T = B * S
NSLOT = 2 * T

BT_R = 512
LANES = 128
NEG = -1e30

BLK = 256
P = NSLOT + E * BLK
NB = P // BLK

NC, NS = 2, 16
NW = NC * NS
L = 16


def _router_body(x_ref, g_ref, w1_ref, w2_ref, i1t_ref, i2t_ref,
                 r1t_ref, r2t_ref, cnt_ref, carry_ref):
    t = pl.program_id(0)
    x = x_ref[...]
    gw = g_ref[...]
    logits = lax.dot_general(x, gw, (((1,), (1,)), ((), ())),
                             preferred_element_type=jnp.float32)
    lane = lax.broadcasted_iota(jnp.int32, (BT_R, LANES), 1)
    l = jnp.where(lane < E, logits, NEG)
    l1 = jnp.max(l, axis=1, keepdims=True)
    i1 = jnp.min(jnp.where(l == l1, lane, LANES), axis=1, keepdims=True)
    lm = jnp.where(lane == i1, NEG, l)
    l2 = jnp.max(lm, axis=1, keepdims=True)
    i2 = jnp.min(jnp.where(lm == l2, lane, LANES), axis=1, keepdims=True)
    w1 = jax.nn.sigmoid(l1 - l2)
    w2 = jax.nn.sigmoid(l2 - l1)

    zc = jnp.zeros((BT_R, LANES), jnp.float32)
    w1_ref[...] = w1 + zc
    w2_ref[...] = w2 + zc

    oh1 = (lane == i1).astype(jnp.float32)
    oh2 = (lane == i2).astype(jnp.float32)
    H = oh1 + oh2

    @pl.when(t == 0)
    def _():
        carry_ref[...] = jnp.zeros_like(carry_ref)

    carr = carry_ref[0:1, :]
    row = lax.broadcasted_iota(jnp.int32, (BT_R, BT_R), 0)
    col = lax.broadcasted_iota(jnp.int32, (BT_R, BT_R), 1)
    Ls = jnp.where(col < row, 1.0, 0.0)
    Cx = lax.dot_general(Ls, H, (((1,), (0,)), ((), ())),
                         preferred_element_type=jnp.float32)
    Cx = Cx + carr
    r1 = jnp.sum(Cx * oh1, axis=1, keepdims=True)
    r2 = jnp.sum(Cx * oh2, axis=1, keepdims=True)

    Ieye = jnp.where(row == col, 1.0, 0.0)
    def tr(v):
        return lax.dot_general(v, Ieye, (((0,), (0,)), ((), ())),
                               precision=lax.Precision.HIGHEST,
                               preferred_element_type=jnp.float32)
    z8 = jnp.zeros((8, BT_R), jnp.float32)
    i1t_ref[...] = (tr(i1.astype(jnp.float32)) + z8).astype(jnp.int32)
    i2t_ref[...] = (tr(i2.astype(jnp.float32)) + z8).astype(jnp.int32)
    r1t_ref[...] = (tr(r1) + z8).astype(jnp.int32)
    r2t_ref[...] = (tr(r2) + z8).astype(jnp.int32)

    new_carry = carr + jnp.sum(H, axis=0, keepdims=True)
    carry_ref[...] = new_carry + jnp.zeros((8, LANES), jnp.float32)
    cnt_ref[...] = new_carry + jnp.zeros((8, LANES), jnp.float32)


def _run_router(x, gate_pad):
    n = T // BT_R
    f32, i32 = jnp.float32, jnp.int32
    outs = jax.ShapeDtypeStruct
    return pl.pallas_call(
        _router_body,
        grid=(n,),
        in_specs=[
            pl.BlockSpec((BT_R, D), lambda t: (t, 0)),
            pl.BlockSpec((LANES, D), lambda t: (0, 0)),
        ],
        out_specs=[
            pl.BlockSpec((BT_R, LANES), lambda t: (t, 0)),
            pl.BlockSpec((BT_R, LANES), lambda t: (t, 0)),
            pl.BlockSpec((8, BT_R), lambda t: (0, t)),
            pl.BlockSpec((8, BT_R), lambda t: (0, t)),
            pl.BlockSpec((8, BT_R), lambda t: (0, t)),
            pl.BlockSpec((8, BT_R), lambda t: (0, t)),
            pl.BlockSpec((8, LANES), lambda t: (0, 0)),
        ],
        out_shape=[
            outs((T, LANES), f32), outs((T, LANES), f32),
            outs((8, T), i32), outs((8, T), i32),
            outs((8, T), i32), outs((8, T), i32),
            outs((8, LANES), f32),
        ],
        scratch_shapes=[pltpu.VMEM((8, LANES), f32)],
        compiler_params=pltpu.CompilerParams(
            dimension_semantics=("arbitrary",)),
    )(x, gate_pad)


SLOTS_PER_W = NSLOT // NW
CH_G = 64
N_IT_G = SLOTS_PER_W // CH_G


def _gather_half(x_hbm, dest_hbm, w_hbm, xs_hbm, ws_hbm,
                 wbuf, destv, rows, sem, base, off0):
    def it(i, c):
        toff = base + i * CH_G
        pltpu.sync_copy(dest_hbm.at[pl.ds(off0 + toff, CH_G)], destv)
        pltpu.sync_copy(w_hbm.at[pl.ds(toff, CH_G)], wbuf)
        pltpu.sync_copy(x_hbm.at[pl.ds(toff, CH_G)], rows)
        pltpu.async_copy(rows, xs_hbm.at[destv], sem).wait()
        pltpu.async_copy(wbuf, ws_hbm.at[destv], sem).wait()
        return c

    lax.fori_loop(0, N_IT_G, it, 0)


def _gather_body(x_hbm, dest_hbm, w1_hbm, w2_hbm, xs_hbm, ws_hbm,
                 wbuf, destv, rows, sem):
    wid = lax.axis_index("s") * NC + lax.axis_index("c")
    base = (wid % (NW // 2)) * SLOTS_PER_W

    @pl.when(wid < NW // 2)
    def _():
        _gather_half(x_hbm, dest_hbm, w1_hbm, xs_hbm, ws_hbm,
                     wbuf, destv, rows, sem, base, 0)

    @pl.when(wid >= NW // 2)
    def _():
        _gather_half(x_hbm, dest_hbm, w2_hbm, xs_hbm, ws_hbm,
                     wbuf, destv, rows, sem, base, T)


def _run_gather(x, dest_all, w1, w2):
    f32, i32 = jnp.float32, jnp.int32
    mesh = plsc.VectorSubcoreMesh(core_axis_name="c", subcore_axis_name="s",
                                  num_cores=NC, num_subcores=NS)
    return pl.kernel(
        _gather_body,
        mesh=mesh,
        out_type=[jax.ShapeDtypeStruct((P, D), f32),
                  jax.ShapeDtypeStruct((P, 128), f32)],
        scratch_types=[
            pltpu.VMEM((CH_G, 128), f32),
            pltpu.VMEM((CH_G,), i32),
            pltpu.VMEM((CH_G, D), f32),
            pltpu.SemaphoreType.DMA,
        ],
    )(x, dest_all, w1, w2)


def _ffn_body(nbu_ref, be_ref, xs_ref, ws_ref, wg_ref, wu_ref, wd_ref, o_ref):
    b = pl.program_id(0)

    @pl.when(b < nbu_ref[0])
    def _():
        xb = xs_ref[...].astype(jnp.bfloat16)
        wg = wg_ref[0]
        wu = wu_ref[0]
        wd = wd_ref[0]
        g = lax.dot_general(xb, wg, (((1,), (1,)), ((), ())),
                            preferred_element_type=jnp.float32)
        u = lax.dot_general(xb, wu, (((1,), (1,)), ((), ())),
                            preferred_element_type=jnp.float32)
        p = (g * jax.nn.sigmoid(g) * u).astype(jnp.bfloat16)
        o = lax.dot_general(p, wd, (((1,), (1,)), ((), ())),
                            preferred_element_type=jnp.float32)
        o_ref[...] = ws_ref[:, 0:1] * o


def _run_ffn(nbu, block_expert, xs, ws, Wg, Wu, Wd):
    grid_spec = pltpu.PrefetchScalarGridSpec(
        num_scalar_prefetch=2,
        grid=(NB,),
        in_specs=[
            pl.BlockSpec((BLK, D), lambda b, nbu, be: (b, 0)),
            pl.BlockSpec((BLK, 128), lambda b, nbu, be: (b, 0)),
            pl.BlockSpec((1, F, D), lambda b, nbu, be: (be[b], 0, 0)),
            pl.BlockSpec((1, F, D), lambda b, nbu, be: (be[b], 0, 0)),
            pl.BlockSpec((1, D, F), lambda b, nbu, be: (be[b], 0, 0)),
        ],
        out_specs=pl.BlockSpec((BLK, D), lambda b, nbu, be: (b, 0)),
    )
    return pl.pallas_call(
        _ffn_body,
        grid_spec=grid_spec,
        out_shape=jax.ShapeDtypeStruct((P, D), jnp.float32),
        compiler_params=pltpu.CompilerParams(
            dimension_semantics=("arbitrary",)),
    )(nbu, block_expert, xs, ws, Wg, Wu, Wd)


TOK_PER_W = T // NW
CH_C = 32
NVEC = D // L


def _combine_body(ys_hbm, dest_hbm, o_hbm, p0v, p1v, bufa, bufb, bufo, sem):
    wid = lax.axis_index("s") * NC + lax.axis_index("c")
    base = wid * TOK_PER_W

    def chunk(i, c):
        off = base + i * CH_C
        pltpu.sync_copy(dest_hbm.at[pl.ds(off, CH_C)], p0v)
        pltpu.sync_copy(dest_hbm.at[pl.ds(T + off, CH_C)], p1v)
        pltpu.async_copy(ys_hbm.at[p0v], bufa, sem).wait()
        pltpu.async_copy(ys_hbm.at[p1v], bufb, sem).wait()

        def rowloop(r, c2):
            for j in range(NVEC):
                a = bufa[r, pl.ds(j * L, L)]
                b = bufb[r, pl.ds(j * L, L)]
                bufo[r, pl.ds(j * L, L)] = a + b
            return c2

        lax.fori_loop(0, CH_C, rowloop, 0)
        pltpu.sync_copy(bufo, o_hbm.at[pl.ds(off, CH_C)])
        return c

    lax.fori_loop(0, TOK_PER_W // CH_C, chunk, 0)


def _run_combine(ys, dest_all):
    f32, i32 = jnp.float32, jnp.int32
    mesh = plsc.VectorSubcoreMesh(core_axis_name="c", subcore_axis_name="s",
                                  num_cores=NC, num_subcores=NS)
    return pl.kernel(
        _combine_body,
        mesh=mesh,
        out_type=jax.ShapeDtypeStruct((T, D), f32),
        scratch_types=[
            pltpu.VMEM((CH_C,), i32),
            pltpu.VMEM((CH_C,), i32),
            pltpu.VMEM((CH_C, D), f32),
            pltpu.VMEM((CH_C, D), f32),
            pltpu.VMEM((CH_C, D), f32),
            pltpu.SemaphoreType.DMA,
        ],
    )(ys, dest_all)


def kernel(hidden_states, gate_w, Wg, Wu, Wd):
    i32 = jnp.int32
    x = hidden_states.reshape(T, D)
    gate_pad = jnp.zeros((LANES, D), jnp.float32).at[:E].set(gate_w)
    w1, w2, i1, i2, r1, r2, cnt = _run_router(x, gate_pad)

    cntv = cnt[0, :E].astype(i32)
    cpad = ((cntv + BLK - 1) // BLK) * BLK
    offs = jnp.concatenate([jnp.zeros((1,), i32),
                            jnp.cumsum(cpad)[:-1].astype(i32)])
    offs_b = offs // BLK
    nbu = (jnp.sum(cpad) // BLK).astype(i32).reshape(1)
    bidx = jnp.arange(NB, dtype=i32)
    block_expert = (jnp.sum((bidx[:, None] >= offs_b[None, :]).astype(i32),
                            axis=1) - 1).astype(i32)

    e_all = jnp.concatenate([i1[0], i2[0]])
    rank_all = jnp.concatenate([r1[0], r2[0]])
    oh = (e_all[:, None] == jnp.arange(E, dtype=i32)[None, :]).astype(i32)
    dest_all = (jnp.sum(oh * offs[None, :], axis=1) + rank_all).astype(i32)

    xs, ws = _run_gather(x, dest_all, w1, w2)
    ys = _run_ffn(nbu, block_expert, xs, ws,
                  Wg.astype(jnp.bfloat16), Wu.astype(jnp.bfloat16),
                  Wd.astype(jnp.bfloat16))
    out = _run_combine(ys, dest_all)
    return out.reshape(B, S, D)
